# baseline (speedup 1.0000x reference)
import functools

import jax
import jax.numpy as jnp
import numpy as np
from jax import lax
from jax.experimental import pallas as pl
from jax.experimental.pallas import tpu as pltpu

_LABEL_SMOOTHING = 0.1
_V = 100000
_B = 1024
_IGNORE = 1
_S = np.float32(_LABEL_SMOOTHING / (_V - 2))
_C = np.float32(1.0 - _LABEL_SMOOTHING)
_K = float(_V - 2) * float(np.float32(_S * np.float32(np.log(_S)))) + float(
    np.float32(_C * np.float32(np.log(_C)))
)

_BR = 1024
_NBF = _V // _BR
_TW = _V - _NBF * _BR
_NB = _NBF + 1

_NBUF = 3
_GB = 32
_NBATCH = _B // _GB


def _body(t_sref, x_any, tv_ref, o_ref, blk_ref, gbuf_ref, acc_ref, x1_ref,
          blksem, gsem):
    j = pl.program_id(0)

    def _blk_copy(jj, rows):
        s = lax.rem(jj, _NBUF)
        start = pl.multiple_of(jj * _BR, _BR)
        return pltpu.make_async_copy(
            x_any.at[pl.ds(start, rows), :],
            blk_ref.at[pl.ds(s * _BR, rows), :],
            blksem.at[s],
        )

    def _gdma(b, k):
        t = t_sref[b * _GB + k]
        vstart = pl.multiple_of((t // 8) * 8, 8)
        lstart = pl.multiple_of((b // 4) * 128, 128)
        off = lax.rem(b, 2) * _GB * 8 + k * 8
        return pltpu.make_async_copy(
            x_any.at[pl.ds(vstart, 8), pl.ds(lstart, 128)],
            gbuf_ref.at[pl.ds(off, 8), :],
            gsem.at[lax.rem(b, 2) * _GB + k],
        )

    @pl.when(j == 0)
    def _init():
        o_ref[...] = jnp.zeros_like(o_ref)
        acc_ref[...] = jnp.zeros_like(acc_ref)
        _blk_copy(0, _BR).start()
        _blk_copy(1, _BR).start()

    @pl.when(j + 2 < _NBF)
    def _prefetch_full():
        _blk_copy(j + 2, _BR).start()

    @pl.when(j + 2 == _NBF)
    def _prefetch_tail():
        _blk_copy(j + 2, _TW).start()

    @pl.when(j < _NBATCH)
    def _gfire():
        for k in range(_GB):
            _gdma(j, k).start()

    @pl.when(jnp.logical_and(j >= 1, j <= _NBATCH))
    def _gdrain():
        b = j - 1
        for k in range(_GB):
            _gdma(b, k).wait()
        rows8 = lax.broadcasted_iota(jnp.int32, (8, 128), 0)
        lanes = lax.broadcasted_iota(jnp.int32, (8, 128), 1)
        part = jnp.zeros((8, 128), jnp.float32)
        for k in range(_GB):
            t = t_sref[b * _GB + k]
            off = lax.rem(b, 2) * _GB * 8 + k * 8
            xw = gbuf_ref[pl.ds(off, 8), :]
            sel = jnp.logical_and(
                jnp.logical_and(rows8 == lax.rem(t, 8),
                                lanes == lax.rem(b, 4) * _GB + k),
                t != _IGNORE,
            )
            part = part + jnp.where(sel, xw, 0.0)
        o_ref[...] += (-(_C - _S) * jnp.sum(part)).reshape(1, 1)

    def _accum(rows):
        s = lax.rem(j, _NBUF)
        xblk = blk_ref[pl.ds(s * _BR, rows), :]
        part = xblk[0:8, :]
        for k in range(1, rows // 8):
            part = part + xblk[k * 8 : (k + 1) * 8, :]
        acc_ref[...] += part
        return xblk

    @pl.when(j < _NBF)
    def _stream_full():
        _blk_copy(j, _BR).wait()
        xblk = _accum(_BR)

        @pl.when(j == 0)
        def _grab_x1():
            x1_ref[...] = xblk[_IGNORE : _IGNORE + 1, :]

    @pl.when(j == _NBF)
    def _stream_tail():
        _blk_copy(j, _TW).wait()
        _accum(_TW)
        rowsum = jnp.sum(acc_ref[...], axis=0, keepdims=True)
        maskf = (tv_ref[...] != _IGNORE).astype(jnp.float32)
        o_ref[...] += (
            _K * jnp.sum(maskf)
            + _S * jnp.sum(maskf * (x1_ref[...] - rowsum))
        ).reshape(1, 1)


@functools.partial(jax.jit)
def kernel(output, target):
    t32 = target.astype(jnp.int32)
    res = pl.pallas_call(
        _body,
        grid_spec=pltpu.PrefetchScalarGridSpec(
            num_scalar_prefetch=1,
            grid=(_NB,),
            in_specs=[
                pl.BlockSpec(memory_space=pl.ANY),
                pl.BlockSpec((1, _B), lambda j, t: (0, 0)),
            ],
            out_specs=pl.BlockSpec((1, 1), lambda j, t: (0, 0)),
            scratch_shapes=[
                pltpu.VMEM((_NBUF * _BR, _B), jnp.float32),
                pltpu.VMEM((2 * _GB * 8, 128), jnp.float32),
                pltpu.VMEM((8, _B), jnp.float32),
                pltpu.VMEM((1, _B), jnp.float32),
                pltpu.SemaphoreType.DMA((_NBUF,)),
                pltpu.SemaphoreType.DMA((2 * _GB,)),
            ],
        ),
        out_shape=jax.ShapeDtypeStruct((1, 1), jnp.float32),
    )(t32, jnp.swapaxes(output, 0, 1), t32.reshape(1, _B))
    return res[0, 0]


# speedup vs baseline: 1.0575x; 1.0575x over previous
import functools

import jax
import jax.numpy as jnp
import numpy as np
from jax import lax
from jax.experimental import pallas as pl
from jax.experimental.pallas import tpu as pltpu

_LABEL_SMOOTHING = 0.1
_V = 100000
_B = 1024
_IGNORE = 1
_S = np.float32(_LABEL_SMOOTHING / (_V - 2))
_C = np.float32(1.0 - _LABEL_SMOOTHING)
_K = float(_V - 2) * float(np.float32(_S * np.float32(np.log(_S)))) + float(
    np.float32(_C * np.float32(np.log(_C)))
)

_BR = 2048
_NBF = _V // _BR
_TW = _V - _NBF * _BR
_NB = _NBF + 1

_NBUF = 3
_GB = 32
_NBATCH = _B // _GB


def _body(t_sref, x_any, tv_ref, o_ref, blk_ref, gbuf_ref, acc_ref, x1_ref,
          blksem, gsem):
    j = pl.program_id(0)

    def _blk_copy(jj, rows):
        s = lax.rem(jj, _NBUF)
        start = pl.multiple_of(jj * _BR, _BR)
        return pltpu.make_async_copy(
            x_any.at[pl.ds(start, rows), :],
            blk_ref.at[pl.ds(s * _BR, rows), :],
            blksem.at[s],
        )

    def _gdma(b, k):
        t = t_sref[b * _GB + k]
        vstart = pl.multiple_of((t // 8) * 8, 8)
        lstart = pl.multiple_of((b // 4) * 128, 128)
        off = lax.rem(b, 2) * _GB * 8 + k * 8
        return pltpu.make_async_copy(
            x_any.at[pl.ds(vstart, 8), pl.ds(lstart, 128)],
            gbuf_ref.at[pl.ds(off, 8), :],
            gsem.at[lax.rem(b, 2) * _GB + k],
        )

    @pl.when(j == 0)
    def _init():
        o_ref[...] = jnp.zeros_like(o_ref)
        acc_ref[...] = jnp.zeros_like(acc_ref)
        _blk_copy(0, _BR).start()
        _blk_copy(1, _BR).start()

    @pl.when(j + 2 < _NBF)
    def _prefetch_full():
        _blk_copy(j + 2, _BR).start()

    @pl.when(j + 2 == _NBF)
    def _prefetch_tail():
        _blk_copy(j + 2, _TW).start()

    @pl.when(j < _NBATCH)
    def _gfire():
        for k in range(_GB):
            _gdma(j, k).start()

    @pl.when(jnp.logical_and(j >= 1, j <= _NBATCH))
    def _gdrain():
        b = j - 1
        for k in range(_GB):
            _gdma(b, k).wait()
        rows8 = lax.broadcasted_iota(jnp.int32, (8, 128), 0)
        lanes = lax.broadcasted_iota(jnp.int32, (8, 128), 1)
        part = jnp.zeros((8, 128), jnp.float32)
        for k in range(_GB):
            t = t_sref[b * _GB + k]
            off = lax.rem(b, 2) * _GB * 8 + k * 8
            xw = gbuf_ref[pl.ds(off, 8), :]
            sel = jnp.logical_and(
                jnp.logical_and(rows8 == lax.rem(t, 8),
                                lanes == lax.rem(b, 4) * _GB + k),
                t != _IGNORE,
            )
            part = part + jnp.where(sel, xw, 0.0)
        o_ref[...] += (-(_C - _S) * jnp.sum(part)).reshape(1, 1)

    def _accum(rows):
        s = lax.rem(j, _NBUF)
        xblk = blk_ref[pl.ds(s * _BR, rows), :]
        part = xblk[0:8, :]
        for k in range(1, rows // 8):
            part = part + xblk[k * 8 : (k + 1) * 8, :]
        acc_ref[...] += part
        return xblk

    @pl.when(j < _NBF)
    def _stream_full():
        _blk_copy(j, _BR).wait()
        xblk = _accum(_BR)

        @pl.when(j == 0)
        def _grab_x1():
            x1_ref[...] = xblk[_IGNORE : _IGNORE + 1, :]

    @pl.when(j == _NBF)
    def _stream_tail():
        _blk_copy(j, _TW).wait()
        _accum(_TW)
        rowsum = jnp.sum(acc_ref[...], axis=0, keepdims=True)
        maskf = (tv_ref[...] != _IGNORE).astype(jnp.float32)
        o_ref[...] += (
            _K * jnp.sum(maskf)
            + _S * jnp.sum(maskf * (x1_ref[...] - rowsum))
        ).reshape(1, 1)


@functools.partial(jax.jit)
def kernel(output, target):
    t32 = target.astype(jnp.int32)
    res = pl.pallas_call(
        _body,
        grid_spec=pltpu.PrefetchScalarGridSpec(
            num_scalar_prefetch=1,
            grid=(_NB,),
            in_specs=[
                pl.BlockSpec(memory_space=pl.ANY),
                pl.BlockSpec((1, _B), lambda j, t: (0, 0)),
            ],
            out_specs=pl.BlockSpec((1, 1), lambda j, t: (0, 0)),
            scratch_shapes=[
                pltpu.VMEM((_NBUF * _BR, _B), jnp.float32),
                pltpu.VMEM((2 * _GB * 8, 128), jnp.float32),
                pltpu.VMEM((8, _B), jnp.float32),
                pltpu.VMEM((1, _B), jnp.float32),
                pltpu.SemaphoreType.DMA((_NBUF,)),
                pltpu.SemaphoreType.DMA((2 * _GB,)),
            ],
        ),
        out_shape=jax.ShapeDtypeStruct((1, 1), jnp.float32),
    )(t32, jnp.swapaxes(output, 0, 1), t32.reshape(1, _B))
    return res[0, 0]
